# baseline (speedup 1.0000x reference)
import functools
import math

import jax
import jax.numpy as jnp
from jax import lax
from jax.experimental import pallas as pl
from jax.experimental.pallas import tpu as pltpu
from jax.experimental.pallas import tpu_sc as plsc

EMBED = 64
ROWS = 95
ROWS_PAD = 96
NC = 2
NS = 16
NW = NC * NS
CHUNK = 128
FB = 32


def _table_body(re_ref, im_ref, wt_ref, b_ref, g_ref, bt_ref, out_ref):
    acc = jnp.zeros((ROWS_PAD, EMBED), jnp.float32)
    for s in range(4):
        r = re_ref[s]
        i = im_ref[s]
        acc = acc + r * r + i * i
    nrm = jnp.sqrt(jnp.sum(acc, axis=1, keepdims=True))
    emb = jnp.sqrt(acc / ((nrm + 1e-8) ** 2) + 1e-12)
    out = jnp.dot(emb, wt_ref[...], preferred_element_type=jnp.float32)
    out = out + b_ref[...]
    mu = jnp.mean(out, axis=1, keepdims=True)
    xc = out - mu
    var = jnp.mean(xc * xc, axis=1, keepdims=True)
    tab = xc * lax.rsqrt(var + 1e-5) * g_ref[...] + bt_ref[...]
    out_ref[...] = jnp.concatenate(
        [tab, jnp.zeros((ROWS_PAD, EMBED), jnp.float32)], axis=1)


def _finalize_body(in_ref, out_ref):
    out_ref[...] = in_ref[:, :EMBED].reshape(out_ref.shape)


@functools.lru_cache(maxsize=None)
def _make_gather(BL: int):
    per_w = BL // NW
    T = per_w // CHUNK
    mesh = plsc.VectorSubcoreMesh(core_axis_name="c", subcore_axis_name="s")

    NBUF = 5
    assert T % NBUF == 0
    scratch = [pltpu.VMEM((T, CHUNK), jnp.int32)]
    scratch += [pltpu.VMEM((CHUNK, 2 * EMBED), jnp.float32) for _ in range(NBUF)]
    scratch += [pltpu.SemaphoreType.DMA for _ in range(NBUF)]

    @functools.partial(
        pl.kernel,
        mesh=mesh,
        out_type=jax.ShapeDtypeStruct((BL, 2 * EMBED), jnp.float32),
        scratch_types=scratch,
        compiler_params=pltpu.CompilerParams(use_tc_tiling_on_sc=True),
    )
    def gather_kernel(table_hbm, idx_hbm, out_hbm, idx_v, *bufsem):
        bufs = bufsem[:NBUF]
        sems = bufsem[NBUF:NBUF * 2]
        wid = lax.axis_index("s") * NC + lax.axis_index("c")
        base = wid * per_w

        pltpu.sync_copy(idx_hbm.at[wid], idx_v)
        for k in range(NBUF):
            pltpu.async_copy(table_hbm.at[idx_v.at[k]], bufs[k], sems[k])

        def body(i, carry):
            g = i * NBUF
            for k in range(NBUF):
                t = g + k
                pltpu.make_async_copy(
                    table_hbm.at[idx_v.at[t]], bufs[k], sems[k]).wait()
                pltpu.sync_copy(bufs[k], out_hbm.at[pl.ds(base + t * CHUNK, CHUNK)])

                @pl.when(t + NBUF < T)
                def _():
                    pltpu.async_copy(
                        table_hbm.at[idx_v.at[t + NBUF]], bufs[k], sems[k])

            return carry

        lax.fori_loop(0, T // NBUF, body, 0)

    return gather_kernel


def kernel(indices, W, b, ln_gamma, ln_beta, theta, base_re, base_im):
    Bq, L = indices.shape
    BL = Bq * L
    re_t = jnp.pad(jnp.transpose(base_re, (2, 0, 1)),
                   ((0, 0), (0, ROWS_PAD - ROWS), (0, 0)))
    im_t = jnp.pad(jnp.transpose(base_im, (2, 0, 1)),
                   ((0, 0), (0, ROWS_PAD - ROWS), (0, 0)))
    table = pl.pallas_call(
        _table_body,
        out_shape=jax.ShapeDtypeStruct((ROWS_PAD, 2 * EMBED), jnp.float32),
    )(re_t, im_t, W.T, b.reshape(1, EMBED),
      ln_gamma.reshape(1, EMBED), ln_beta.reshape(1, EMBED))

    idx3 = indices.reshape(-1).astype(jnp.int32).reshape(NW, BL // (NW * CHUNK), CHUNK)
    flat = _make_gather(BL)(table, idx3)

    out = pl.pallas_call(
        _finalize_body,
        grid=(Bq // FB,),
        in_specs=[pl.BlockSpec((FB * L, 2 * EMBED), lambda i: (i, 0))],
        out_specs=pl.BlockSpec((FB, L, EMBED), lambda i: (i, 0, 0)),
        out_shape=jax.ShapeDtypeStruct((Bq, L, EMBED), jnp.float32),
    )(flat)
    return out


# speedup vs baseline: 2.4709x; 2.4709x over previous
import functools
import math

import jax
import jax.numpy as jnp
from jax import lax
from jax.experimental import pallas as pl
from jax.experimental.pallas import tpu as pltpu
from jax.experimental.pallas import tpu_sc as plsc

EMBED = 64
ROWS = 95
ROWS_PAD = 96
NC = 2
NS = 16
NW = NC * NS
LPAD = 56


def _table_body(re_ref, im_ref, wt_ref, b_ref, g_ref, bt_ref, out_ref):
    acc = jnp.zeros((ROWS_PAD, EMBED), jnp.float32)
    for s in range(4):
        r = re_ref[s]
        i = im_ref[s]
        acc = acc + r * r + i * i
    nrm = jnp.sqrt(jnp.sum(acc, axis=1, keepdims=True))
    emb = jnp.sqrt(acc / ((nrm + 1e-8) ** 2) + 1e-12)
    out = jnp.dot(emb, wt_ref[...], preferred_element_type=jnp.float32)
    out = out + b_ref[...]
    mu = jnp.mean(out, axis=1, keepdims=True)
    xc = out - mu
    var = jnp.mean(xc * xc, axis=1, keepdims=True)
    out_ref[...] = xc * lax.rsqrt(var + 1e-5) * g_ref[...] + bt_ref[...]


@functools.lru_cache(maxsize=None)
def _make_gather(Bq: int, L: int):
    per_w = Bq // NW
    flat_w = per_w * LPAD
    mesh = plsc.VectorSubcoreMesh(core_axis_name="c", subcore_axis_name="s")

    NBUF = 4
    assert per_w % NBUF == 0
    scratch = [pltpu.VMEM((flat_w,), jnp.int32)]
    scratch += [pltpu.VMEM((LPAD, EMBED), jnp.float32) for _ in range(NBUF)]
    scratch += [pltpu.SemaphoreType.DMA for _ in range(NBUF)]
    scratch += [pltpu.VMEM_SHARED((ROWS_PAD, EMBED), jnp.float32)]

    @functools.partial(
        pl.kernel,
        mesh=mesh,
        out_type=jax.ShapeDtypeStruct((Bq, L, EMBED), jnp.float32),
        scratch_types=scratch,
        compiler_params=pltpu.CompilerParams(use_tc_tiling_on_sc=False),
    )
    def gather_kernel(table_hbm, idx_hbm, out_hbm, idx_v, *bufsem):
        bufs = bufsem[:NBUF]
        sems = bufsem[NBUF:NBUF * 2]
        tab_sh = bufsem[NBUF * 2]
        sid = lax.axis_index("s")
        wid = sid * NC + lax.axis_index("c")
        base_b = wid * per_w

        @pl.when(sid == 0)
        def _():
            pltpu.sync_copy(table_hbm, tab_sh)

        pltpu.sync_copy(idx_hbm.at[pl.ds(wid * flat_w, flat_w)], idx_v)
        plsc.subcore_barrier()

        def idx_slice(t):
            return idx_v.at[pl.ds(pl.multiple_of(t * LPAD, 8), LPAD)]

        for k in range(NBUF):
            pltpu.async_copy(tab_sh.at[idx_slice(k)], bufs[k], sems[k])

        def body(i, carry):
            g = i * NBUF
            for k in range(NBUF):
                t = g + k
                pltpu.make_async_copy(
                    tab_sh.at[idx_slice(t)], bufs[k], sems[k]).wait()
                pltpu.sync_copy(bufs[k].at[pl.ds(0, L)], out_hbm.at[base_b + t])

                @pl.when(t + NBUF < per_w)
                def _():
                    pltpu.async_copy(tab_sh.at[idx_slice(t + NBUF)],
                                     bufs[k], sems[k])

            return carry

        lax.fori_loop(0, per_w // NBUF, body, 0)

    return gather_kernel


def kernel(indices, W, b, ln_gamma, ln_beta, theta, base_re, base_im):
    Bq, L = indices.shape
    re_t = jnp.pad(jnp.transpose(base_re, (2, 0, 1)),
                   ((0, 0), (0, ROWS_PAD - ROWS), (0, 0)))
    im_t = jnp.pad(jnp.transpose(base_im, (2, 0, 1)),
                   ((0, 0), (0, ROWS_PAD - ROWS), (0, 0)))
    table = pl.pallas_call(
        _table_body,
        out_shape=jax.ShapeDtypeStruct((ROWS_PAD, EMBED), jnp.float32),
    )(re_t, im_t, W.T, b.reshape(1, EMBED),
      ln_gamma.reshape(1, EMBED), ln_beta.reshape(1, EMBED))

    idxp = jnp.pad(indices.astype(jnp.int32), ((0, 0), (0, LPAD - L)))
    return _make_gather(Bq, L)(table, idxp.reshape(-1))


# speedup vs baseline: 2.5386x; 1.0274x over previous
import functools
import math

import jax
import jax.numpy as jnp
from jax import lax
from jax.experimental import pallas as pl
from jax.experimental.pallas import tpu as pltpu
from jax.experimental.pallas import tpu_sc as plsc

EMBED = 64
ROWS = 95
ROWS_PAD = 96
NC = 2
NS = 16
NW = NC * NS
CHUNK = 128


def _table_body(re_ref, im_ref, wt_ref, b_ref, g_ref, bt_ref, out_ref):
    acc = jnp.zeros((ROWS_PAD, EMBED), jnp.float32)
    for s in range(4):
        r = re_ref[s]
        i = im_ref[s]
        acc = acc + r * r + i * i
    nrm = jnp.sqrt(jnp.sum(acc, axis=1, keepdims=True))
    emb = jnp.sqrt(acc / ((nrm + 1e-8) ** 2) + 1e-12)
    out = jnp.dot(emb, wt_ref[...], preferred_element_type=jnp.float32)
    out = out + b_ref[...]
    mu = jnp.mean(out, axis=1, keepdims=True)
    xc = out - mu
    var = jnp.mean(xc * xc, axis=1, keepdims=True)
    out_ref[...] = xc * lax.rsqrt(var + 1e-5) * g_ref[...] + bt_ref[...]


@functools.lru_cache(maxsize=None)
def _make_gather(BL: int):
    per_w = BL // NW
    T = per_w // CHUNK
    mesh = plsc.VectorSubcoreMesh(core_axis_name="c", subcore_axis_name="s")

    NBUF = 5
    assert T % NBUF == 0
    scratch = [pltpu.VMEM((T, CHUNK), jnp.int32)]
    scratch += [pltpu.VMEM((CHUNK, EMBED), jnp.float32) for _ in range(NBUF)]
    scratch += [pltpu.SemaphoreType.DMA for _ in range(NBUF)]
    scratch += [pltpu.VMEM_SHARED((ROWS_PAD, EMBED), jnp.float32)]

    @functools.partial(
        pl.kernel,
        mesh=mesh,
        out_type=jax.ShapeDtypeStruct((BL, EMBED), jnp.float32),
        scratch_types=scratch,
        compiler_params=pltpu.CompilerParams(use_tc_tiling_on_sc=False),
    )
    def gather_kernel(table_hbm, idx_hbm, out_hbm, idx_v, *bufsem):
        bufs = bufsem[:NBUF]
        sems = bufsem[NBUF:NBUF * 2]
        tab_sh = bufsem[NBUF * 2]
        wid = lax.axis_index("s") * NC + lax.axis_index("c")
        base = wid * per_w

        @pl.when(lax.axis_index("s") == 0)
        def _():
            pltpu.sync_copy(table_hbm, tab_sh)

        pltpu.sync_copy(idx_hbm.at[wid], idx_v)
        plsc.subcore_barrier()
        for k in range(NBUF):
            pltpu.async_copy(tab_sh.at[idx_v.at[k]], bufs[k], sems[k])

        def body(i, carry):
            g = i * NBUF
            for k in range(NBUF):
                t = g + k
                pltpu.make_async_copy(
                    tab_sh.at[idx_v.at[t]], bufs[k], sems[k]).wait()
                pltpu.sync_copy(bufs[k], out_hbm.at[pl.ds(base + t * CHUNK, CHUNK)])

                @pl.when(t + NBUF < T)
                def _():
                    pltpu.async_copy(
                        tab_sh.at[idx_v.at[t + NBUF]], bufs[k], sems[k])

            return carry

        lax.fori_loop(0, T // NBUF, body, 0)

    return gather_kernel


def kernel(indices, W, b, ln_gamma, ln_beta, theta, base_re, base_im):
    Bq, L = indices.shape
    BL = Bq * L
    re_t = jnp.pad(jnp.transpose(base_re, (2, 0, 1)),
                   ((0, 0), (0, ROWS_PAD - ROWS), (0, 0)))
    im_t = jnp.pad(jnp.transpose(base_im, (2, 0, 1)),
                   ((0, 0), (0, ROWS_PAD - ROWS), (0, 0)))
    table = pl.pallas_call(
        _table_body,
        out_shape=jax.ShapeDtypeStruct((ROWS_PAD, EMBED), jnp.float32),
    )(re_t, im_t, W.T, b.reshape(1, EMBED),
      ln_gamma.reshape(1, EMBED), ln_beta.reshape(1, EMBED))

    idx2 = indices.reshape(-1).astype(jnp.int32).reshape(NW, BL // (NW * CHUNK), CHUNK)
    flat = _make_gather(BL)(table, idx2)
    return flat.reshape(Bq, L, EMBED)
